# Initial kernel scaffold; baseline (speedup 1.0000x reference)
import jax
import jax.numpy as jnp
from jax.experimental import pallas as pl


def kernel(h, A, W, b, bag_label):
    raise NotImplementedError("write your pallas kernel here")


# speedup vs baseline: 1.1766x; 1.1766x over previous
import functools

import jax
import jax.numpy as jnp
from jax import lax
from jax.experimental import pallas as pl
from jax.experimental.pallas import tpu as pltpu
from jax.experimental.pallas import tpu_sc as plsc

N = 20000
D = 512
N_INS = 8
LANES = 16
NTILES = 16
CHUNK = 1280
NPAD = NTILES * CHUNK
NV_FULL = CHUNK // LANES
NV_LAST = (N - (NTILES - 1) * CHUNK) // LANES
BIG_I = 2 ** 30
SENT = -2.0


def _sc_body(a_hbm, h_hbm, wt_hbm, b_hbm, out_hbm,
             a_v, st_f, st_i, spm_f, spm_i, cand_f, cand_i,
             idx_v, rows_v, wt_v, b_v, out_v, sem):
    cid = lax.axis_index("c")
    sid = lax.axis_index("s")
    iota = lax.iota(jnp.int32, LANES)
    base = sid * CHUNK

    pltpu.sync_copy(a_hbm.at[pl.ds(base, CHUNK)], a_v)

    sgn = jnp.where(cid == 0, jnp.float32(1.0), jnp.float32(-1.0))
    nv = jnp.where(sid == NTILES - 1, NV_LAST, NV_FULL)

    def pick_rounds(n_rounds, nvregs, load_fn, lane_off):
        def round_body(r, st):
            selv, seli, pv, pi = st

            def scan_body(j, st2):
                m, mi = st2
                v, gi = load_fn(j)
                elig = (v < pv) | ((v == pv) & (gi > pi))
                veff = jnp.where(elig, v, jnp.float32(SENT))
                upd = veff > m
                return (jnp.where(upd, veff, m), jnp.where(upd, gi, mi))

            m0 = jnp.full((LANES,), SENT, jnp.float32)
            i0 = jnp.full((LANES,), BIG_I, jnp.int32)
            m, mi = lax.fori_loop(0, nvregs, scan_body, (m0, i0))
            mval = jnp.max(m)
            midx = jnp.min(jnp.where(m == mval, mi, BIG_I))
            selv = jnp.where(iota == r + lane_off, mval, selv)
            seli = jnp.where(iota == r + lane_off, midx, seli)
            return (selv, seli, mval, midx)

        st0 = (jnp.full((LANES,), SENT, jnp.float32),
               jnp.zeros((LANES,), jnp.int32),
               jnp.float32(2.0), jnp.int32(-1))
        selv, seli, _, _ = lax.fori_loop(0, n_rounds, round_body, st0)
        return selv, seli

    def load_local(j):
        v = a_v[pl.ds(j * LANES, LANES)] * sgn
        gi = base + j * LANES + iota
        return v, gi

    selv, seli = pick_rounds(N_INS, nv, load_local, 0)

    st_f[...] = selv
    st_i[...] = seli
    pltpu.sync_copy(st_f, spm_f.at[pl.ds(sid * LANES, LANES)])
    pltpu.sync_copy(st_i, spm_i.at[pl.ds(sid * LANES, LANES)])
    plsc.subcore_barrier()

    @pl.when(sid == 0)
    def _():
        pltpu.sync_copy(spm_f, cand_f)
        pltpu.sync_copy(spm_i, cand_i)

        def load_cand(j):
            return (cand_f[pl.ds(j * LANES, LANES)],
                    cand_i[pl.ds(j * LANES, LANES)])

        _, gsel = pick_rounds(N_INS, NTILES, load_cand, 0)

        idx_v[...] = jnp.where(iota < N_INS, gsel, 0)
        pltpu.async_copy(h_hbm.at[idx_v], rows_v, sem).wait()

        pltpu.sync_copy(wt_hbm, wt_v)
        pltpu.sync_copy(b_hbm, b_v)

        l0 = jnp.zeros((LANES,), jnp.float32)
        l1 = jnp.zeros((LANES,), jnp.float32)
        for i in range(N_INS):
            def mm_body(j, acc):
                a0, a1 = acc
                rv = rows_v[i, pl.ds(j * LANES, LANES)]
                a0 = a0 + rv * wt_v[0, pl.ds(j * LANES, LANES)]
                a1 = a1 + rv * wt_v[1, pl.ds(j * LANES, LANES)]
                return (a0, a1)

            z = jnp.zeros((LANES,), jnp.float32)
            a0, a1 = lax.fori_loop(0, D // LANES, mm_body, (z, z))
            l0 = jnp.where(iota == i, jnp.sum(a0), l0)
            l1 = jnp.where(iota == i, jnp.sum(a1), l1)

        l0 = l0 + b_v[0, :]
        l1 = l1 + b_v[1, :]
        p0 = 1.0 / (1.0 + jnp.exp(l1 - l0))
        p1 = 1.0 / (1.0 + jnp.exp(l0 - l1))
        out_v[0, :] = p0
        out_v[1, :] = p1
        pltpu.sync_copy(out_v, out_hbm.at[cid])


@jax.jit
def _sc_call(a_pad, h2, wt, b2):
    mesh = plsc.VectorSubcoreMesh(core_axis_name="c", subcore_axis_name="s")
    fn = pl.kernel(
        _sc_body,
        mesh=mesh,
        out_type=jax.ShapeDtypeStruct((2, 2, LANES), jnp.float32),
        compiler_params=pltpu.CompilerParams(needs_layout_passes=False),
        scratch_types=[
            pltpu.VMEM((CHUNK,), jnp.float32),
            pltpu.VMEM((LANES,), jnp.float32),
            pltpu.VMEM((LANES,), jnp.int32),
            pltpu.VMEM_SHARED((NTILES * LANES,), jnp.float32),
            pltpu.VMEM_SHARED((NTILES * LANES,), jnp.int32),
            pltpu.VMEM((NTILES * LANES,), jnp.float32),
            pltpu.VMEM((NTILES * LANES,), jnp.int32),
            pltpu.VMEM((LANES,), jnp.int32),
            pltpu.VMEM((LANES, D), jnp.float32),
            pltpu.VMEM((2, D), jnp.float32),
            pltpu.VMEM((2, LANES), jnp.float32),
            pltpu.VMEM((2, LANES), jnp.float32),
            pltpu.SemaphoreType.DMA,
        ],
    )
    return fn(a_pad, h2, wt, b2)


def kernel(h, A, W, b, bag_label):
    A_I = A[:, 0, bag_label]
    a_pad = jnp.concatenate(
        [A_I, jnp.zeros((NPAD - N,), jnp.float32)])
    h2 = h.reshape(N, D)
    wt = W.T
    b2 = jnp.broadcast_to(b[:, None], (2, LANES))
    out = _sc_call(a_pad, h2, wt, b2)
    top = out[0, :, :N_INS].T
    bot = out[1, :, :N_INS].T
    logits = jnp.concatenate([top, bot], axis=0).reshape(2 * N_INS, 1, 2)
    labels = jnp.concatenate([jnp.ones((N_INS,), jnp.int32),
                              jnp.zeros((N_INS,), jnp.int32)])
    return labels, logits


# speedup vs baseline: 2.4055x; 2.0444x over previous
import functools

import jax
import jax.numpy as jnp
from jax import lax
from jax.experimental import pallas as pl
from jax.experimental.pallas import tpu as pltpu
from jax.experimental.pallas import tpu_sc as plsc

N = 20000
D = 512
N_INS = 8
LANES = 16
NTILES = 16
CHUNK = 1280
NPAD = NTILES * CHUNK
NV_FULL = CHUNK // LANES
NV_LAST = (N - (NTILES - 1) * CHUNK) // LANES
BIG_I = 2 ** 30
SENT = -2.0


def _sc_body(a_hbm, h_hbm, wt_hbm, b_hbm, out_hbm,
             a_v, st_f, st_i, spm_f, spm_i, cand_f, cand_i,
             idx_v, rows_v, wt_v, b_v, out_v, sem):
    cid = lax.axis_index("c")
    sid = lax.axis_index("s")
    iota = lax.iota(jnp.int32, LANES)
    base = sid * CHUNK

    pltpu.sync_copy(a_hbm.at[pl.ds(base, CHUNK)], a_v)

    sgn = jnp.where(cid == 0, jnp.float32(1.0), jnp.float32(-1.0))
    nv = jnp.where(sid == NTILES - 1, NV_LAST, NV_FULL)

    def pick_rounds(n_rounds, nvregs, load_fn, lane_off):
        def round_body(r, st):
            selv, seli, pv, pi = st

            def scan_body(j, st2):
                m, mi = st2
                v, gi = load_fn(j)
                elig = (v < pv) | ((v == pv) & (gi > pi))
                veff = jnp.where(elig, v, jnp.float32(SENT))
                upd = veff > m
                return (jnp.where(upd, veff, m), jnp.where(upd, gi, mi))

            m0 = jnp.full((LANES,), SENT, jnp.float32)
            i0 = jnp.full((LANES,), BIG_I, jnp.int32)
            m, mi = lax.fori_loop(0, nvregs, scan_body, (m0, i0))
            mval = jnp.max(m)
            midx = jnp.min(jnp.where(m == mval, mi, BIG_I))
            selv = jnp.where(iota == r + lane_off, mval, selv)
            seli = jnp.where(iota == r + lane_off, midx, seli)
            return (selv, seli, mval, midx)

        st0 = (jnp.full((LANES,), SENT, jnp.float32),
               jnp.zeros((LANES,), jnp.int32),
               jnp.float32(2.0), jnp.int32(-1))
        selv, seli, _, _ = lax.fori_loop(0, n_rounds, round_body, st0)
        return selv, seli

    def load_local(j):
        v = a_v[pl.ds(j * LANES, LANES)] * sgn
        gi = base + j * LANES + iota
        return v, gi

    selv, seli = pick_rounds(N_INS, nv, load_local, 0)

    st_f[...] = selv
    st_i[...] = seli
    pltpu.sync_copy(st_f, spm_f.at[pl.ds(sid * LANES, LANES)])
    pltpu.sync_copy(st_i, spm_i.at[pl.ds(sid * LANES, LANES)])
    plsc.subcore_barrier()

    @pl.when(sid == 0)
    def _():
        pltpu.sync_copy(spm_f, cand_f)
        pltpu.sync_copy(spm_i, cand_i)

        def load_cand(j):
            return (cand_f[pl.ds(j * LANES, LANES)],
                    cand_i[pl.ds(j * LANES, LANES)])

        _, gsel = pick_rounds(N_INS, NTILES, load_cand, 0)

        idx_v[...] = jnp.where(iota < N_INS, gsel, 0)
        pltpu.async_copy(h_hbm.at[idx_v], rows_v, sem).wait()

        pltpu.sync_copy(wt_hbm, wt_v)
        pltpu.sync_copy(b_hbm, b_v)

        l0 = jnp.zeros((LANES,), jnp.float32)
        l1 = jnp.zeros((LANES,), jnp.float32)
        for i in range(N_INS):
            def mm_body(j, acc):
                a0, a1 = acc
                rv = rows_v[i, pl.ds(j * LANES, LANES)]
                a0 = a0 + rv * wt_v[0, pl.ds(j * LANES, LANES)]
                a1 = a1 + rv * wt_v[1, pl.ds(j * LANES, LANES)]
                return (a0, a1)

            z = jnp.zeros((LANES,), jnp.float32)
            a0, a1 = lax.fori_loop(0, D // LANES, mm_body, (z, z))
            l0 = jnp.where(iota == i, jnp.sum(a0), l0)
            l1 = jnp.where(iota == i, jnp.sum(a1), l1)

        l0 = l0 + b_v[0, :]
        l1 = l1 + b_v[1, :]
        p0 = 1.0 / (1.0 + jnp.exp(l1 - l0))
        p1 = 1.0 / (1.0 + jnp.exp(l0 - l1))
        out_v[0, :] = p0
        out_v[1, :] = p1
        pltpu.sync_copy(out_v, out_hbm.at[cid])


@jax.jit
def _sc_call(a_pad, h2, wt, b2):
    mesh = plsc.VectorSubcoreMesh(core_axis_name="c", subcore_axis_name="s")
    fn = pl.kernel(
        _sc_body,
        mesh=mesh,
        out_type=jax.ShapeDtypeStruct((2, 2, LANES), jnp.float32),
        compiler_params=pltpu.CompilerParams(
            needs_layout_passes=False, use_tc_tiling_on_sc=False),
        scratch_types=[
            pltpu.VMEM((CHUNK,), jnp.float32),
            pltpu.VMEM((LANES,), jnp.float32),
            pltpu.VMEM((LANES,), jnp.int32),
            pltpu.VMEM_SHARED((NTILES * LANES,), jnp.float32),
            pltpu.VMEM_SHARED((NTILES * LANES,), jnp.int32),
            pltpu.VMEM((NTILES * LANES,), jnp.float32),
            pltpu.VMEM((NTILES * LANES,), jnp.int32),
            pltpu.VMEM((LANES,), jnp.int32),
            pltpu.VMEM((LANES, D), jnp.float32),
            pltpu.VMEM((2, D), jnp.float32),
            pltpu.VMEM((2, LANES), jnp.float32),
            pltpu.VMEM((2, LANES), jnp.float32),
            pltpu.SemaphoreType.DMA,
        ],
    )
    return fn(a_pad, h2, wt, b2)


def kernel(h, A, W, b, bag_label):
    A_I = A[:, 0, bag_label]
    a_pad = jnp.concatenate(
        [A_I, jnp.zeros((NPAD - N,), jnp.float32)])
    h2 = h.reshape(N, D)
    wt = W.T
    b2 = jnp.broadcast_to(b[:, None], (2, LANES))
    out = _sc_call(a_pad, h2, wt, b2)
    top = out[0, :, :N_INS].T
    bot = out[1, :, :N_INS].T
    logits = jnp.concatenate([top, bot], axis=0).reshape(2 * N_INS, 1, 2)
    labels = jnp.concatenate([jnp.ones((N_INS,), jnp.int32),
                              jnp.zeros((N_INS,), jnp.int32)])
    return labels, logits


# speedup vs baseline: 2.5765x; 1.0711x over previous
import jax
import jax.numpy as jnp
from jax import lax
from jax.experimental import pallas as pl
from jax.experimental.pallas import tpu as pltpu
from jax.experimental.pallas import tpu_sc as plsc

N = 20000
D = 512
N_INS = 8
LANES = 16
NTILES = 16
CHUNK = 1280
NPAD = NTILES * CHUNK
INNER = 8
OUTER = CHUNK // (LANES * INNER)
BIG_I = 2 ** 30
SENT = -2.0


def _insert_step(v, gi, ms, mi):
    c = [v > m for m in ms]
    nm = [jnp.where(c[0], v, ms[0])]
    ni = [jnp.where(c[0], gi, mi[0])]
    for k in range(1, N_INS):
        nm.append(jnp.where(c[k], jnp.where(c[k - 1], ms[k - 1], v), ms[k]))
        ni.append(jnp.where(c[k], jnp.where(c[k - 1], mi[k - 1], gi), mi[k]))
    return nm, ni


def _extract8(ms, mi, iota):
    def round_body(r, st):
        selv, seli, pv, pi = st
        m = jnp.full((LANES,), SENT, jnp.float32)
        ii = jnp.full((LANES,), BIG_I, jnp.int32)
        for k in range(N_INS):
            v, gi = ms[k], mi[k]
            elig = (v < pv) | ((v == pv) & (gi > pi))
            veff = jnp.where(elig, v, jnp.float32(SENT))
            upd = veff > m
            m = jnp.where(upd, veff, m)
            ii = jnp.where(upd, gi, ii)
        mval = jnp.max(m)
        midx = jnp.min(jnp.where(m == mval, ii, BIG_I))
        selv = jnp.where(iota == r, mval, selv)
        seli = jnp.where(iota == r, midx, seli)
        return (selv, seli, mval, midx)

    st0 = (jnp.full((LANES,), SENT, jnp.float32),
           jnp.zeros((LANES,), jnp.int32),
           jnp.float32(2.0), jnp.int32(-1))
    selv, seli, _, _ = lax.fori_loop(0, N_INS, round_body, st0)
    return selv, seli


def _sc_body(a_hbm, h_hbm, wtb_hbm, probs_hbm, lab_hbm,
             a_v, st_f, st_i, spm_f, spm_i, cand_f, cand_i,
             idx_v, rows_v, wtb_v, out_v, lab_v, sem):
    cid = lax.axis_index("c")
    sid = lax.axis_index("s")
    iota = lax.iota(jnp.int32, LANES)
    base = sid * CHUNK

    pltpu.sync_copy(a_hbm.at[pl.ds(base, CHUNK)], a_v)

    sgn = jnp.where(cid == 0, jnp.float32(1.0), jnp.float32(-1.0))

    def scan_body(j, st):
        ms, mi = list(st[0]), list(st[1])
        off0 = j * (LANES * INNER)
        for k in range(INNER):
            off = off0 + k * LANES
            v = a_v[pl.ds(off, LANES)] * sgn
            gi = base + off + iota
            ms, mi = _insert_step(v, gi, ms, mi)
        return (tuple(ms), tuple(mi))

    st0 = (tuple(jnp.full((LANES,), SENT, jnp.float32) for _ in range(N_INS)),
           tuple(jnp.full((LANES,), BIG_I, jnp.int32) for _ in range(N_INS)))
    ms, mi = lax.fori_loop(0, OUTER, scan_body, st0)
    selv, seli = _extract8(list(ms), list(mi), iota)

    st_f[...] = selv
    st_i[...] = seli
    pltpu.sync_copy(st_f, spm_f.at[pl.ds(sid * LANES, LANES)])
    pltpu.sync_copy(st_i, spm_i.at[pl.ds(sid * LANES, LANES)])
    plsc.subcore_barrier()

    @pl.when(sid == 0)
    def _():
        pltpu.sync_copy(spm_f, cand_f)
        pltpu.sync_copy(spm_i, cand_i)

        ms = [jnp.full((LANES,), SENT, jnp.float32) for _ in range(N_INS)]
        mi = [jnp.full((LANES,), BIG_I, jnp.int32) for _ in range(N_INS)]
        for t in range(NTILES):
            v = cand_f[pl.ds(t * LANES, LANES)]
            gi = cand_i[pl.ds(t * LANES, LANES)]
            ms, mi = _insert_step(v, gi, ms, mi)
        _, gsel = _extract8(ms, mi, iota)

        idx_v[...] = jnp.where(iota < N_INS, gsel, 0)
        pltpu.async_copy(h_hbm.at[idx_v], rows_v, sem).wait()

        pltpu.sync_copy(wtb_hbm, wtb_v)

        l0 = jnp.zeros((LANES,), jnp.float32)
        l1 = jnp.zeros((LANES,), jnp.float32)
        for i in range(N_INS):
            a0 = jnp.zeros((LANES,), jnp.float32)
            a1 = jnp.zeros((LANES,), jnp.float32)
            for j in range(D // LANES):
                rv = rows_v[i, 0, pl.ds(j * LANES, LANES)]
                a0 = a0 + rv * wtb_v[0, pl.ds(j * LANES, LANES)]
                a1 = a1 + rv * wtb_v[1, pl.ds(j * LANES, LANES)]
            l0 = jnp.where(iota == i, jnp.sum(a0), l0)
            l1 = jnp.where(iota == i, jnp.sum(a1), l1)

        bvec = wtb_v[0, pl.ds(D, LANES)]
        bvec1 = wtb_v[1, pl.ds(D, LANES)]
        l0 = l0 + bvec[0]
        l1 = l1 + bvec1[0]
        p0 = 1.0 / (1.0 + jnp.exp(l1 - l0))
        p1 = 1.0 / (1.0 + jnp.exp(l0 - l1))
        lane_ok = iota < N_INS
        plsc.store_scatter(out_v, [2 * iota], p0, mask=lane_ok)
        plsc.store_scatter(out_v, [2 * iota + 1], p1, mask=lane_ok)
        pltpu.sync_copy(out_v, probs_hbm.at[pl.ds(cid * 2 * N_INS, 2 * N_INS)])

        lab_v[...] = jnp.broadcast_to(1 - cid, (LANES,)).astype(jnp.int32)
        pltpu.sync_copy(lab_v.at[pl.ds(0, N_INS)],
                        lab_hbm.at[pl.ds(cid * N_INS, N_INS)])


@jax.jit
def _sc_call(a_pad, h, wtb):
    mesh = plsc.VectorSubcoreMesh(core_axis_name="c", subcore_axis_name="s")
    fn = pl.kernel(
        _sc_body,
        mesh=mesh,
        out_type=[jax.ShapeDtypeStruct((4 * N_INS,), jnp.float32),
                  jax.ShapeDtypeStruct((2 * N_INS,), jnp.int32)],
        compiler_params=pltpu.CompilerParams(
            needs_layout_passes=False, use_tc_tiling_on_sc=False),
        scratch_types=[
            pltpu.VMEM((CHUNK,), jnp.float32),
            pltpu.VMEM((LANES,), jnp.float32),
            pltpu.VMEM((LANES,), jnp.int32),
            pltpu.VMEM_SHARED((NTILES * LANES,), jnp.float32),
            pltpu.VMEM_SHARED((NTILES * LANES,), jnp.int32),
            pltpu.VMEM((NTILES * LANES,), jnp.float32),
            pltpu.VMEM((NTILES * LANES,), jnp.int32),
            pltpu.VMEM((LANES,), jnp.int32),
            pltpu.VMEM((LANES, 1, D), jnp.float32),
            pltpu.VMEM((2, D + LANES), jnp.float32),
            pltpu.VMEM((2 * N_INS,), jnp.float32),
            pltpu.VMEM((LANES,), jnp.int32),
            pltpu.SemaphoreType.DMA,
        ],
    )
    return fn(a_pad, h, wtb)


def kernel(h, A, W, b, bag_label):
    A_I = A[:, 0, bag_label]
    a_pad = jnp.concatenate(
        [A_I, jnp.full((NPAD - N,), jnp.nan, jnp.float32)])
    wtb = jnp.concatenate(
        [W.T, jnp.broadcast_to(b[:, None], (2, LANES))], axis=1)
    probs_flat, labels = _sc_call(a_pad, h, wtb)
    logits = probs_flat.reshape(2 * N_INS, 1, 2)
    return labels, logits


# speedup vs baseline: 2.6597x; 1.0323x over previous
import jax
import jax.numpy as jnp
from jax import lax
from jax.experimental import pallas as pl
from jax.experimental.pallas import tpu as pltpu
from jax.experimental.pallas import tpu_sc as plsc

N = 20000
D = 512
N_INS = 8
LANES = 16
NTILES = 16
CHUNK = 1280
LAST_BASE = N - CHUNK
LAST_LO = (NTILES - 1) * CHUNK
INNER = 4
OUTER = CHUNK // (LANES * INNER)
BIG_I = 2 ** 30
SENT = -2.0
MASKED = -3.0


def _insert_step(v, gi, ms, mi):
    c = [v > m for m in ms]
    nm = [jnp.where(c[0], v, ms[0])]
    ni = [jnp.where(c[0], gi, mi[0])]
    for k in range(1, N_INS):
        nm.append(jnp.where(c[k], jnp.where(c[k - 1], ms[k - 1], v), ms[k]))
        ni.append(jnp.where(c[k], jnp.where(c[k - 1], mi[k - 1], gi), mi[k]))
    return nm, ni


def _extract8(ms, mi, iota):
    def round_body(r, st):
        selv, seli, pv, pi = st
        m = jnp.full((LANES,), SENT, jnp.float32)
        ii = jnp.full((LANES,), BIG_I, jnp.int32)
        for k in range(N_INS):
            v, gi = ms[k], mi[k]
            elig = (v < pv) | ((v == pv) & (gi > pi))
            veff = jnp.where(elig, v, jnp.float32(SENT))
            upd = veff > m
            m = jnp.where(upd, veff, m)
            ii = jnp.where(upd, gi, ii)
        mval = jnp.max(m)
        midx = jnp.min(jnp.where(m == mval, ii, BIG_I))
        selv = jnp.where(iota == r, mval, selv)
        seli = jnp.where(iota == r, midx, seli)
        return (selv, seli, mval, midx)

    st0 = (jnp.full((LANES,), SENT, jnp.float32),
           jnp.zeros((LANES,), jnp.int32),
           jnp.float32(2.0), jnp.int32(-1))
    selv, seli, _, _ = lax.fori_loop(0, N_INS, round_body, st0)
    return selv, seli


def _fresh_state():
    return ([jnp.full((LANES,), SENT, jnp.float32) for _ in range(N_INS)],
            [jnp.full((LANES,), BIG_I, jnp.int32) for _ in range(N_INS)])


def _sc_body(a_hbm, h_hbm, w_hbm, b_hbm, probs_hbm, lab_hbm,
             a_v, st_f, st_i, spm_f, spm_i, cand_f, cand_i,
             idx_v, rows_v, w_v, b_v, o0_v, o1_v, lab_v, sem):
    cid = lax.axis_index("c")
    sid = lax.axis_index("s")
    iota = lax.iota(jnp.int32, LANES)
    last = sid == NTILES - 1
    base = jnp.where(last, LAST_BASE, sid * CHUNK)
    lo = jnp.where(last, LAST_LO, 0)

    pltpu.sync_copy(a_hbm.at[pl.ds(base, CHUNK)], a_v)

    sgn = jnp.where(cid == 0, jnp.float32(1.0), jnp.float32(-1.0))
    bi = base + iota

    def scan_body(j, st):
        ms, mi = list(st[0]), list(st[1])
        off0 = j * (LANES * INNER)
        for k in range(INNER):
            off = off0 + k * LANES
            gi = bi + off
            v = a_v[pl.ds(off, LANES)] * sgn
            v = jnp.where(gi >= lo, v, jnp.float32(MASKED))
            ms, mi = _insert_step(v, gi, ms, mi)
        return (tuple(ms), tuple(mi))

    ms0, mi0 = _fresh_state()
    ms, mi = lax.fori_loop(0, OUTER, scan_body, (tuple(ms0), tuple(mi0)))
    selv, seli = _extract8(list(ms), list(mi), iota)

    st_f[...] = selv
    st_i[...] = seli
    pltpu.sync_copy(st_f, spm_f.at[pl.ds(sid * LANES, LANES)])
    pltpu.sync_copy(st_i, spm_i.at[pl.ds(sid * LANES, LANES)])
    plsc.subcore_barrier()

    @pl.when(sid == 0)
    def _():
        pltpu.sync_copy(spm_f, cand_f)
        pltpu.sync_copy(spm_i, cand_i)

        def merge_body(t, st):
            ms, mi = list(st[0]), list(st[1])
            v = cand_f[pl.ds(t * LANES, LANES)]
            gi = cand_i[pl.ds(t * LANES, LANES)]
            ms, mi = _insert_step(v, gi, ms, mi)
            return (tuple(ms), tuple(mi))

        ms0, mi0 = _fresh_state()
        ms, mi = lax.fori_loop(0, NTILES, merge_body, (tuple(ms0), tuple(mi0)))
        _, gsel = _extract8(list(ms), list(mi), iota)

        idx_v[...] = jnp.where(iota < N_INS, gsel, 0)
        pltpu.async_copy(h_hbm.at[idx_v], rows_v, sem).wait()

        pltpu.sync_copy(w_hbm, w_v)
        pltpu.sync_copy(b_hbm, b_v.at[pl.ds(0, 2)])

        def mm_body(j, st):
            a0s, a1s = list(st[0]), list(st[1])
            wt0 = w_v[pl.ds(j * LANES, LANES)]
            wt1 = w_v[pl.ds(D + j * LANES, LANES)]
            for i in range(N_INS):
                rv = rows_v[i, 0, pl.ds(j * LANES, LANES)]
                a0s[i] = a0s[i] + rv * wt0
                a1s[i] = a1s[i] + rv * wt1
            return (tuple(a0s), tuple(a1s))

        z = [jnp.zeros((LANES,), jnp.float32) for _ in range(N_INS)]
        a0s, a1s = lax.fori_loop(0, D // LANES, mm_body,
                                 (tuple(z), tuple(z)))
        l0 = jnp.zeros((LANES,), jnp.float32)
        l1 = jnp.zeros((LANES,), jnp.float32)
        for i in range(N_INS):
            l0 = jnp.where(iota == i, jnp.sum(a0s[i]), l0)
            l1 = jnp.where(iota == i, jnp.sum(a1s[i]), l1)

        bv = b_v[...]
        l0 = l0 + bv[0]
        l1 = l1 + bv[1]
        o0_v[...] = 1.0 / (1.0 + jnp.exp(l1 - l0))
        o1_v[...] = 1.0 / (1.0 + jnp.exp(l0 - l1))
        pltpu.sync_copy(o0_v.at[pl.ds(0, N_INS)],
                        probs_hbm.at[pl.ds(cid * N_INS, N_INS)])
        pltpu.sync_copy(o1_v.at[pl.ds(0, N_INS)],
                        probs_hbm.at[pl.ds(2 * N_INS + cid * N_INS, N_INS)])

        lab_v[...] = jnp.broadcast_to(1 - cid, (LANES,)).astype(jnp.int32)
        pltpu.sync_copy(lab_v.at[pl.ds(0, N_INS)],
                        lab_hbm.at[pl.ds(cid * N_INS, N_INS)])


@jax.jit
def _sc_call(a_i, h, wflat, b):
    mesh = plsc.VectorSubcoreMesh(core_axis_name="c", subcore_axis_name="s")
    fn = pl.kernel(
        _sc_body,
        mesh=mesh,
        out_type=[jax.ShapeDtypeStruct((4 * N_INS,), jnp.float32),
                  jax.ShapeDtypeStruct((2 * N_INS,), jnp.int32)],
        compiler_params=pltpu.CompilerParams(
            needs_layout_passes=False, use_tc_tiling_on_sc=False),
        scratch_types=[
            pltpu.VMEM((CHUNK,), jnp.float32),
            pltpu.VMEM((LANES,), jnp.float32),
            pltpu.VMEM((LANES,), jnp.int32),
            pltpu.VMEM_SHARED((NTILES * LANES,), jnp.float32),
            pltpu.VMEM_SHARED((NTILES * LANES,), jnp.int32),
            pltpu.VMEM((NTILES * LANES,), jnp.float32),
            pltpu.VMEM((NTILES * LANES,), jnp.int32),
            pltpu.VMEM((LANES,), jnp.int32),
            pltpu.VMEM((LANES, 1, D), jnp.float32),
            pltpu.VMEM((2 * D,), jnp.float32),
            pltpu.VMEM((LANES,), jnp.float32),
            pltpu.VMEM((LANES,), jnp.float32),
            pltpu.VMEM((LANES,), jnp.float32),
            pltpu.VMEM((LANES,), jnp.int32),
            pltpu.SemaphoreType.DMA,
        ],
    )
    return fn(a_i, h, wflat, b)


def kernel(h, A, W, b, bag_label):
    a_i = A[:, 0, bag_label]
    wflat = W.T.reshape(2 * D)
    probs_flat, labels = _sc_call(a_i, h, wflat, b)
    logits = probs_flat.reshape(2, 2 * N_INS).T.reshape(2 * N_INS, 1, 2)
    return labels, logits


# speedup vs baseline: 2.7703x; 1.0416x over previous
import jax
import jax.numpy as jnp
from jax import lax
from jax.experimental import pallas as pl
from jax.experimental.pallas import tpu as pltpu
from jax.experimental.pallas import tpu_sc as plsc

N = 20000
D = 512
N_INS = 8
LANES = 16
NTILES = 16
CHUNK = 1280
LAST_BASE = N - CHUNK
LAST_LO = (NTILES - 1) * CHUNK
INNER = 4
OUTER = CHUNK // (LANES * INNER)
BIG_I = 2 ** 30
SENT = -2.0
MASKED = -3.0


def _insert_step(v, gi, ms, mi):
    c = [v > m for m in ms]
    nm = [jnp.where(c[0], v, ms[0])]
    ni = [jnp.where(c[0], gi, mi[0])]
    for k in range(1, N_INS):
        nm.append(jnp.where(c[k], jnp.where(c[k - 1], ms[k - 1], v), ms[k]))
        ni.append(jnp.where(c[k], jnp.where(c[k - 1], mi[k - 1], gi), mi[k]))
    return nm, ni


def _extract8(ms, mi, iota):
    def round_body(r, st):
        selv, seli, pv, pi = st
        m = jnp.full((LANES,), SENT, jnp.float32)
        ii = jnp.full((LANES,), BIG_I, jnp.int32)
        for k in range(N_INS):
            v, gi = ms[k], mi[k]
            elig = (v < pv) | ((v == pv) & (gi > pi))
            veff = jnp.where(elig, v, jnp.float32(SENT))
            upd = veff > m
            m = jnp.where(upd, veff, m)
            ii = jnp.where(upd, gi, ii)
        mval = jnp.max(m)
        midx = jnp.min(jnp.where(m == mval, ii, BIG_I))
        selv = jnp.where(iota == r, mval, selv)
        seli = jnp.where(iota == r, midx, seli)
        return (selv, seli, mval, midx)

    st0 = (jnp.full((LANES,), SENT, jnp.float32),
           jnp.zeros((LANES,), jnp.int32),
           jnp.float32(2.0), jnp.int32(-1))
    selv, seli, _, _ = lax.fori_loop(0, N_INS, round_body, st0)
    return selv, seli


def _fresh_state():
    return ([jnp.full((LANES,), SENT, jnp.float32) for _ in range(N_INS)],
            [jnp.full((LANES,), BIG_I, jnp.int32) for _ in range(N_INS)])


WB_OFF = N
WB_LEN = 8 + 2 * D
W_OFF = 8


def _sc_body(pk_hbm, h_hbm, probs_hbm, lab_hbm,
             a_v, st_f, st_i, spm_f, spm_i, cand_f, cand_i,
             idx_v, rows_v, wb_v, o0_v, o1_v, lab_v, sem, sem2):
    cid = lax.axis_index("c")
    sid = lax.axis_index("s")
    iota = lax.iota(jnp.int32, LANES)
    last = sid == NTILES - 1
    base = jnp.where(last, LAST_BASE, sid * CHUNK)
    lo = jnp.where(last, LAST_LO, 0)

    copy_a = pltpu.async_copy(pk_hbm.at[pl.ds(base, CHUNK)], a_v, sem)

    @pl.when(sid == 0)
    def _():
        pltpu.async_copy(pk_hbm.at[pl.ds(WB_OFF, WB_LEN)], wb_v, sem2)

    copy_a.wait()

    sgn = jnp.where(cid == 0, jnp.float32(1.0), jnp.float32(-1.0))
    bi = base + iota

    def scan_body(j, st):
        ms, mi = list(st[0]), list(st[1])
        off0 = j * (LANES * INNER)
        for k in range(INNER):
            off = off0 + k * LANES
            gi = bi + off
            v = a_v[pl.ds(off, LANES)] * sgn
            v = jnp.where(gi >= lo, v, jnp.float32(MASKED))
            ms, mi = _insert_step(v, gi, ms, mi)
        return (tuple(ms), tuple(mi))

    ms0, mi0 = _fresh_state()
    ms, mi = lax.fori_loop(0, OUTER, scan_body, (tuple(ms0), tuple(mi0)))
    selv, seli = _extract8(list(ms), list(mi), iota)

    st_f[...] = selv
    st_i[...] = seli
    pltpu.sync_copy(st_f, spm_f.at[pl.ds(sid * LANES, LANES)])
    pltpu.sync_copy(st_i, spm_i.at[pl.ds(sid * LANES, LANES)])
    plsc.subcore_barrier()

    @pl.when(sid == 0)
    def _():
        pltpu.sync_copy(spm_f, cand_f)
        pltpu.sync_copy(spm_i, cand_i)

        def merge_body(t, st):
            ms, mi = list(st[0]), list(st[1])
            v = cand_f[pl.ds(t * LANES, LANES)]
            gi = cand_i[pl.ds(t * LANES, LANES)]
            ms, mi = _insert_step(v, gi, ms, mi)
            return (tuple(ms), tuple(mi))

        ms0, mi0 = _fresh_state()
        ms, mi = lax.fori_loop(0, NTILES, merge_body, (tuple(ms0), tuple(mi0)))
        _, gsel = _extract8(list(ms), list(mi), iota)

        idx_v[...] = jnp.where(iota < N_INS, gsel, 0)
        pltpu.async_copy(h_hbm.at[idx_v], rows_v, sem).wait()

        pltpu.make_async_copy(pk_hbm.at[pl.ds(WB_OFF, WB_LEN)], wb_v,
                              sem2).wait()

        def mm_body(j, st):
            a0s, a1s = list(st[0]), list(st[1])
            wt0 = wb_v[pl.ds(W_OFF + j * LANES, LANES)]
            wt1 = wb_v[pl.ds(W_OFF + D + j * LANES, LANES)]
            for i in range(N_INS):
                rv = rows_v[i, 0, pl.ds(j * LANES, LANES)]
                a0s[i] = a0s[i] + rv * wt0
                a1s[i] = a1s[i] + rv * wt1
            return (tuple(a0s), tuple(a1s))

        z = [jnp.zeros((LANES,), jnp.float32) for _ in range(N_INS)]
        a0s, a1s = lax.fori_loop(0, D // LANES, mm_body,
                                 (tuple(z), tuple(z)))
        l0 = jnp.zeros((LANES,), jnp.float32)
        l1 = jnp.zeros((LANES,), jnp.float32)
        for i in range(N_INS):
            l0 = jnp.where(iota == i, jnp.sum(a0s[i]), l0)
            l1 = jnp.where(iota == i, jnp.sum(a1s[i]), l1)

        bv = wb_v[pl.ds(0, LANES)]
        l0 = l0 + bv[0]
        l1 = l1 + bv[1]
        o0_v[...] = 1.0 / (1.0 + jnp.exp(l1 - l0))
        o1_v[...] = 1.0 / (1.0 + jnp.exp(l0 - l1))
        pltpu.sync_copy(o0_v.at[pl.ds(0, N_INS)],
                        probs_hbm.at[pl.ds(cid * N_INS, N_INS)])
        pltpu.sync_copy(o1_v.at[pl.ds(0, N_INS)],
                        probs_hbm.at[pl.ds(2 * N_INS + cid * N_INS, N_INS)])

        lab_v[...] = jnp.broadcast_to(1 - cid, (LANES,)).astype(jnp.int32)
        pltpu.sync_copy(lab_v.at[pl.ds(0, N_INS)],
                        lab_hbm.at[pl.ds(cid * N_INS, N_INS)])


@jax.jit
def _sc_call(packed, h):
    mesh = plsc.VectorSubcoreMesh(core_axis_name="c", subcore_axis_name="s")
    fn = pl.kernel(
        _sc_body,
        mesh=mesh,
        out_type=[jax.ShapeDtypeStruct((4 * N_INS,), jnp.float32),
                  jax.ShapeDtypeStruct((2 * N_INS,), jnp.int32)],
        compiler_params=pltpu.CompilerParams(
            needs_layout_passes=False, use_tc_tiling_on_sc=False,
            disable_bounds_checks=True, disable_semaphore_checks=True),
        scratch_types=[
            pltpu.VMEM((CHUNK,), jnp.float32),
            pltpu.VMEM((LANES,), jnp.float32),
            pltpu.VMEM((LANES,), jnp.int32),
            pltpu.VMEM_SHARED((NTILES * LANES,), jnp.float32),
            pltpu.VMEM_SHARED((NTILES * LANES,), jnp.int32),
            pltpu.VMEM((NTILES * LANES,), jnp.float32),
            pltpu.VMEM((NTILES * LANES,), jnp.int32),
            pltpu.VMEM((LANES,), jnp.int32),
            pltpu.VMEM((LANES, 1, D), jnp.float32),
            pltpu.VMEM((8 + 2 * D,), jnp.float32),
            pltpu.VMEM((LANES,), jnp.float32),
            pltpu.VMEM((LANES,), jnp.float32),
            pltpu.VMEM((LANES,), jnp.int32),
            pltpu.SemaphoreType.DMA,
            pltpu.SemaphoreType.DMA,
        ],
    )
    return fn(packed, h)


def kernel(h, A, W, b, bag_label):
    a_i = A[:, 0, bag_label]
    packed = jnp.concatenate(
        [a_i, b, jnp.zeros((6,), jnp.float32), W.T.reshape(2 * D)])
    probs_flat, labels = _sc_call(packed, h)
    logits = probs_flat.reshape(2, 2 * N_INS).T.reshape(2 * N_INS, 1, 2)
    return labels, logits


# speedup vs baseline: 2.7723x; 1.0007x over previous
import jax
import jax.numpy as jnp
from jax import lax
from jax.experimental import pallas as pl
from jax.experimental.pallas import tpu as pltpu
from jax.experimental.pallas import tpu_sc as plsc

N = 20000
D = 512
N_INS = 8
LANES = 16
NTILES = 16
CHUNK = 1280
LAST_BASE = N - CHUNK
LAST_LO = (NTILES - 1) * CHUNK
INNER = 4
OUTER = CHUNK // (LANES * INNER)
BIG_I = 2 ** 30
SENT = -2.0
MASKED = -3.0


def _insert_step(v, gi, ms, mi):
    c = [v > m for m in ms]
    nm = [jnp.where(c[0], v, ms[0])]
    ni = [jnp.where(c[0], gi, mi[0])]
    for k in range(1, N_INS):
        nm.append(jnp.where(c[k], jnp.where(c[k - 1], ms[k - 1], v), ms[k]))
        ni.append(jnp.where(c[k], jnp.where(c[k - 1], mi[k - 1], gi), mi[k]))
    return nm, ni


def _extract8(ms, mi, iota):
    def round_body(r, st):
        selv, seli, pv, pi = st
        m = jnp.full((LANES,), SENT, jnp.float32)
        ii = jnp.full((LANES,), BIG_I, jnp.int32)
        for k in range(N_INS):
            v, gi = ms[k], mi[k]
            elig = (v < pv) | ((v == pv) & (gi > pi))
            veff = jnp.where(elig, v, jnp.float32(SENT))
            upd = veff > m
            m = jnp.where(upd, veff, m)
            ii = jnp.where(upd, gi, ii)
        mval = jnp.max(m)
        midx = jnp.min(jnp.where(m == mval, ii, BIG_I))
        selv = jnp.where(iota == r, mval, selv)
        seli = jnp.where(iota == r, midx, seli)
        return (selv, seli, mval, midx)

    st0 = (jnp.full((LANES,), SENT, jnp.float32),
           jnp.zeros((LANES,), jnp.int32),
           jnp.float32(2.0), jnp.int32(-1))
    selv, seli, _, _ = lax.fori_loop(0, N_INS, round_body, st0)
    return selv, seli


def _fresh_state():
    return ([jnp.full((LANES,), SENT, jnp.float32) for _ in range(N_INS)],
            [jnp.full((LANES,), BIG_I, jnp.int32) for _ in range(N_INS)])


WB_OFF = N
WB_LEN = 8 + 2 * D
W_OFF = 8


def _sc_body(pk_hbm, h_hbm, probs_hbm, lab_hbm,
             a_v, st_f, st_i, spm_f, spm_i, cand_f, cand_i,
             idx_v, rows_v, wb_v, o0_v, o1_v, lab_v, sem, sem2):
    cid = lax.axis_index("c")
    sid = lax.axis_index("s")
    iota = lax.iota(jnp.int32, LANES)
    last = sid == NTILES - 1
    base = jnp.where(last, LAST_BASE, sid * CHUNK)
    lo = jnp.where(last, LAST_LO, 0)

    copy_a = pltpu.async_copy(pk_hbm.at[pl.ds(base, CHUNK)], a_v, sem)

    @pl.when(sid == 0)
    def _():
        pltpu.async_copy(pk_hbm.at[pl.ds(WB_OFF, WB_LEN)], wb_v, sem2)

    copy_a.wait()

    sgn = jnp.where(cid == 0, jnp.float32(1.0), jnp.float32(-1.0))
    bi = base + iota

    def scan_body(j, st):
        ms, mi = list(st[0]), list(st[1])
        off0 = j * (LANES * INNER)
        for k in range(INNER):
            off = off0 + k * LANES
            gi = bi + off
            v = a_v[pl.ds(off, LANES)] * sgn
            v = jnp.where(gi >= lo, v, jnp.float32(MASKED))
            ms, mi = _insert_step(v, gi, ms, mi)
        return (tuple(ms), tuple(mi))

    ms0, mi0 = _fresh_state()
    ms, mi = lax.fori_loop(0, OUTER, scan_body, (tuple(ms0), tuple(mi0)))
    selv, seli = _extract8(list(ms), list(mi), iota)

    st_f[...] = selv
    st_i[...] = seli
    pltpu.sync_copy(st_f, spm_f.at[pl.ds(sid * LANES, LANES)])
    pltpu.sync_copy(st_i, spm_i.at[pl.ds(sid * LANES, LANES)])
    plsc.subcore_barrier()

    @pl.when(sid == 0)
    def _():
        pltpu.sync_copy(spm_f, cand_f)
        pltpu.sync_copy(spm_i, cand_i)

        def merge_body(t, st):
            ms, mi = list(st[0]), list(st[1])
            v = cand_f[pl.ds(t * LANES, LANES)]
            gi = cand_i[pl.ds(t * LANES, LANES)]
            ms, mi = _insert_step(v, gi, ms, mi)
            return (tuple(ms), tuple(mi))

        ms0, mi0 = _fresh_state()
        ms, mi = lax.fori_loop(0, NTILES, merge_body, (tuple(ms0), tuple(mi0)))
        _, gsel = _extract8(list(ms), list(mi), iota)

        idx_v[...] = jnp.where(iota < N_INS, gsel, 0)
        pltpu.async_copy(h_hbm.at[idx_v], rows_v, sem).wait()

        pltpu.make_async_copy(pk_hbm.at[pl.ds(WB_OFF, WB_LEN)], wb_v,
                              sem2).wait()

        def mm_body(j, st):
            a0s, a1s = list(st[0]), list(st[1])
            wt0 = wb_v[pl.ds(W_OFF + j * LANES, LANES)]
            wt1 = wb_v[pl.ds(W_OFF + D + j * LANES, LANES)]
            for i in range(N_INS):
                rv = rows_v[i, 0, pl.ds(j * LANES, LANES)]
                a0s[i] = a0s[i] + rv * wt0
                a1s[i] = a1s[i] + rv * wt1
            return (tuple(a0s), tuple(a1s))

        z = [jnp.zeros((LANES,), jnp.float32) for _ in range(N_INS)]
        a0s, a1s = lax.fori_loop(0, D // LANES, mm_body,
                                 (tuple(z), tuple(z)))
        l0 = jnp.zeros((LANES,), jnp.float32)
        l1 = jnp.zeros((LANES,), jnp.float32)
        for i in range(N_INS):
            l0 = jnp.where(iota == i, jnp.sum(a0s[i]), l0)
            l1 = jnp.where(iota == i, jnp.sum(a1s[i]), l1)

        bv = wb_v[pl.ds(0, LANES)]
        l0 = l0 + bv[0]
        l1 = l1 + bv[1]
        o0_v[...] = 1.0 / (1.0 + jnp.exp(l1 - l0))
        o1_v[...] = 1.0 / (1.0 + jnp.exp(l0 - l1))
        pltpu.sync_copy(o0_v.at[pl.ds(0, N_INS)],
                        probs_hbm.at[pl.ds(cid * N_INS, N_INS)])
        pltpu.sync_copy(o1_v.at[pl.ds(0, N_INS)],
                        probs_hbm.at[pl.ds(2 * N_INS + cid * N_INS, N_INS)])

        lab_v[...] = jnp.broadcast_to(1 - cid, (LANES,)).astype(jnp.int32)
        pltpu.sync_copy(lab_v.at[pl.ds(0, N_INS)],
                        lab_hbm.at[pl.ds(cid * N_INS, N_INS)])


@jax.jit
def _sc_call(packed, h):
    mesh = plsc.VectorSubcoreMesh(core_axis_name="c", subcore_axis_name="s")
    fn = pl.kernel(
        _sc_body,
        mesh=mesh,
        out_type=[jax.ShapeDtypeStruct((4 * N_INS,), jnp.float32),
                  jax.ShapeDtypeStruct((2 * N_INS,), jnp.int32)],
        compiler_params=pltpu.CompilerParams(
            needs_layout_passes=False, use_tc_tiling_on_sc=False,
            disable_bounds_checks=True, disable_semaphore_checks=True,
            skip_device_barrier=True),
        scratch_types=[
            pltpu.VMEM((CHUNK,), jnp.float32),
            pltpu.VMEM((LANES,), jnp.float32),
            pltpu.VMEM((LANES,), jnp.int32),
            pltpu.VMEM_SHARED((NTILES * LANES,), jnp.float32),
            pltpu.VMEM_SHARED((NTILES * LANES,), jnp.int32),
            pltpu.VMEM((NTILES * LANES,), jnp.float32),
            pltpu.VMEM((NTILES * LANES,), jnp.int32),
            pltpu.VMEM((LANES,), jnp.int32),
            pltpu.VMEM((LANES, 1, D), jnp.float32),
            pltpu.VMEM((8 + 2 * D,), jnp.float32),
            pltpu.VMEM((LANES,), jnp.float32),
            pltpu.VMEM((LANES,), jnp.float32),
            pltpu.VMEM((LANES,), jnp.int32),
            pltpu.SemaphoreType.DMA,
            pltpu.SemaphoreType.DMA,
        ],
    )
    return fn(packed, h)


def kernel(h, A, W, b, bag_label):
    a_i = A[:, 0, bag_label]
    packed = jnp.concatenate(
        [a_i, b, jnp.zeros((6,), jnp.float32), W.T.reshape(2 * D)])
    probs_flat, labels = _sc_call(packed, h)
    logits = probs_flat.reshape(2, 2 * N_INS).T.reshape(2 * N_INS, 1, 2)
    return labels, logits
